# baseline (speedup 1.0000x reference)
import jax
import jax.numpy as jnp
from jax import lax
from jax.experimental import pallas as pl
from jax.experimental.pallas import tpu as pltpu
from jax.experimental.pallas import tpu_sc as plsc

N = 10000
R = 32
D = 128
B = 25
E = 320000
NR = N * R

NC = 2
NS = 16
NW = NC * NS
EPW = E // NW

KA = 80
NCHA = EPW // KA
NR_PT = NR // NS

KC = 80
KH = KC // 2
NP = 10240
N_PT = NP // NS
SUP = 5
EPS = EPW // SUP
CPS = EPS // KC

_MESH2 = plsc.VectorSubcoreMesh(core_axis_name="c", subcore_axis_name="s")


def _a1_body(src_hbm, dst_hbm, rel_hbm,
             seg_hbm, rowid_hbm, w_hbm,
             src_v, dst_v, rel_v, segf_v, rowf_v, segc_v, ones_v, zb_v,
             cnt_v, cnt2_v, wf_v, cnt_sh, csem, csem2):
    c = lax.axis_index("c")
    s = lax.axis_index("s")

    def zfill(i, _):
        zb_v[pl.ds(i * 16, 16)] = jnp.zeros((16,), jnp.float32)
        return 0

    lax.fori_loop(0, NR_PT // 16, zfill, 0)
    pltpu.sync_copy(zb_v, cnt_sh.at[pl.ds(s * NR_PT, NR_PT)])
    for j in range(KA // 16):
        ones_v[pl.ds(j * 16, 16)] = jnp.ones((16,), jnp.float32)
    plsc.subcore_barrier()

    for h in range(2):
        rng = s + NS * h
        base = rng * EPW
        pltpu.sync_copy(src_hbm.at[pl.ds(base, EPW)], src_v)
        pltpu.sync_copy(dst_hbm.at[pl.ds(base, EPW)], dst_v)
        pltpu.sync_copy(rel_hbm.at[pl.ds(base, EPW)], rel_v)

        def chunk(ci, _):
            for j in range(KA // 16):
                off = ci * KA + j * 16
                s16 = src_v[pl.ds(off, 16)]
                d16 = dst_v[pl.ds(off, 16)]
                r16 = rel_v[pl.ds(off, 16)]
                seg16 = d16 * R + r16
                row16 = r16 * N + s16
                segc_v[pl.ds(j * 16, 16)] = seg16
                segf_v[pl.ds(off, 16)] = seg16
                rowf_v[pl.ds(off, 16)] = row16
            pltpu.sync_copy(ones_v, cnt_sh.at[segc_v], add=True)
            return 0

        lax.fori_loop(0, NCHA, chunk, 0)

        @pl.when(c == h)
        def _():
            pltpu.sync_copy(segf_v, seg_hbm.at[pl.ds(base, EPW)])
            pltpu.sync_copy(rowf_v, rowid_hbm.at[pl.ds(base, EPW)])

    plsc.subcore_barrier()

    own = (c * NS + s) * EPW
    pltpu.sync_copy(seg_hbm.at[pl.ds(own, EPW)], segf_v)

    cbufs = (cnt_v, cnt2_v)
    csems = (csem, csem2)

    def _wissue(ci, b):
        pltpu.async_copy(cnt_sh.at[segf_v.at[pl.ds(ci * KA, KA)]],
                         cbufs[b], csems[b])

    _wissue(0, 0)
    _wissue(1, 1)

    def wpair(i2, _):
        for b in range(2):
            ci = i2 * 2 + b
            pltpu.make_async_copy(cnt_sh.at[segf_v.at[pl.ds(0, KA)]],
                                  cbufs[b], csems[b]).wait()
            nci = ci + 2

            @pl.when(nci < NCHA)
            def _():
                _wissue(nci, b)

            for j in range(KA // 16):
                cv = cbufs[b][pl.ds(j * 16, 16)]
                wf_v[pl.ds(ci * KA + j * 16, 16)] = 1.0 / cv
        return 0

    lax.fori_loop(0, NCHA // 2, wpair, 0)
    pltpu.make_async_copy(cnt_sh.at[segf_v.at[pl.ds(0, KA)]],
                          cbufs[0], csems[0]).wait()
    for j in range(KA // 16):
        cv = cnt_v[pl.ds(j * 16, 16)]
        wf_v[pl.ds((NCHA - 1) * KA + j * 16, 16)] = 1.0 / cv
    pltpu.sync_copy(wf_v, w_hbm.at[pl.ds(own, EPW)])


_a1 = pl.kernel(
    _a1_body,
    out_type=(
        jax.ShapeDtypeStruct((E,), jnp.int32),
        jax.ShapeDtypeStruct((E,), jnp.int32),
        jax.ShapeDtypeStruct((E,), jnp.float32),
    ),
    mesh=_MESH2,
    scratch_types=(
        pltpu.VMEM((EPW,), jnp.int32),
        pltpu.VMEM((EPW,), jnp.int32),
        pltpu.VMEM((EPW,), jnp.int32),
        pltpu.VMEM((EPW,), jnp.int32),
        pltpu.VMEM((EPW,), jnp.int32),
        pltpu.VMEM((KA,), jnp.int32),
        pltpu.VMEM((KA,), jnp.float32),
        pltpu.VMEM((NR_PT,), jnp.float32),
        pltpu.VMEM((KA,), jnp.float32),
        pltpu.VMEM((KA,), jnp.float32),
        pltpu.VMEM((EPW,), jnp.float32),
        pltpu.VMEM_SHARED((NR,), jnp.float32),
        pltpu.SemaphoreType.DMA,
        pltpu.SemaphoreType.DMA,
    ),
)


def _c_body(t_hbm, rowid_hbm, dst3_hbm, w_hbm, znd_hbm,
            acc_hbm,
            rowid_v, w_v, dst2_v, gb0, gb1, sb0, sb1, acc_sh,
            gsem0, gsem1, ssem0, ssem1):
    c = lax.axis_index("c")
    s = lax.axis_index("s")
    wid = c * NS + s
    base = wid * EPW

    pltpu.sync_copy(znd_hbm.at[pl.ds(s * N_PT, N_PT)],
                    acc_sh.at[pl.ds(s * N_PT, N_PT)])
    plsc.subcore_barrier()

    gbufs = (gb0, gb1)
    gsems = (gsem0, gsem1)
    sbufs = (sb0, sb1)
    ssems = (ssem0, ssem1)

    def _issue(ci, b):
        idx0 = rowid_v.at[pl.ds(ci * KC, KH)]
        idx1 = rowid_v.at[pl.ds(ci * KC + KH, KH)]
        pltpu.async_copy(t_hbm.at[idx0], gbufs[b].at[pl.ds(0, KH)], gsems[b])
        pltpu.async_copy(t_hbm.at[idx1], gbufs[b].at[pl.ds(KH, KH)], gsems[b])

    def _wait_gather(b):
        pltpu.make_async_copy(t_hbm.at[rowid_v.at[pl.ds(0, KC)]],
                              gbufs[b], gsems[b]).wait()

    def _wait_scatter(h):
        pltpu.make_async_copy(sbufs[h], acc_sh.at[dst2_v.at[0]],
                              ssems[h]).wait()

    def _chunk(ci, b, first):
        _wait_gather(b)
        gbuf = gbufs[b]
        cbase = ci * KC
        w16 = tuple(w_v[pl.ds(cbase + 16 * j, 16)] for j in range(KC // 16))
        for h in range(2):
            if not first:
                _wait_scatter(h)
            sbuf = sbufs[h]
            for el in range(KH):
                e = h * KH + el
                j, lane = e // 16, e % 16
                wb = w16[j].at[jnp.full((16,), lane, jnp.int32)].get(
                    mode="promise_in_bounds")
                for k in range(D // 16):
                    sbuf[el, pl.ds(k * 16, 16)] = (
                        gbuf[e, pl.ds(k * 16, 16)] * wb)
            pltpu.async_copy(sbufs[h], acc_sh.at[dst2_v.at[2 * ci + h]],
                             ssems[h], add=True)

        nci = ci + 2

        @pl.when(nci < CPS)
        def _():
            _issue(nci, b)

    def sup(g, _):
        @pl.when(g > 0)
        def _():
            _wait_scatter(0)
            _wait_scatter(1)

        sbase = base + g * EPS
        pltpu.sync_copy(rowid_hbm.at[pl.ds(sbase, EPS)], rowid_v)
        pltpu.sync_copy(w_hbm.at[pl.ds(sbase, EPS)], w_v)
        pltpu.sync_copy(dst3_hbm.at[wid, g], dst2_v)
        _issue(0, 0)
        _issue(1, 1)

        _chunk(0, 0, True)
        _chunk(1, 1, False)

        def pair(i2, _):
            ci0 = i2 * 2 + 2
            _chunk(ci0, 0, False)
            _chunk(ci0 + 1, 1, False)
            return 0

        lax.fori_loop(0, (CPS - 2) // 2, pair, 0)
        _chunk(CPS - 1, 0, False)
        return 0

    lax.fori_loop(0, SUP, sup, 0)
    _wait_scatter(0)
    _wait_scatter(1)
    plsc.subcore_barrier()
    pltpu.sync_copy(acc_sh.at[pl.ds(s * N_PT, N_PT)],
                    acc_hbm.at[c, pl.ds(s * N_PT, N_PT)])


_c = pl.kernel(
    _c_body,
    out_type=jax.ShapeDtypeStruct((NC, NP, D), jnp.float32),
    mesh=_MESH2,
    scratch_types=(
        pltpu.VMEM((EPS,), jnp.int32),
        pltpu.VMEM((EPS,), jnp.float32),
        pltpu.VMEM((2 * CPS, KH), jnp.int32),
        pltpu.VMEM((KC, D), jnp.float32),
        pltpu.VMEM((KC, D), jnp.float32),
        pltpu.VMEM((KH, D), jnp.float32),
        pltpu.VMEM((KH, D), jnp.float32),
        pltpu.VMEM_SHARED((NP, D), jnp.float32),
        pltpu.SemaphoreType.DMA,
        pltpu.SemaphoreType.DMA,
        pltpu.SemaphoreType.DMA,
        pltpu.SemaphoreType.DMA,
    ),
    compiler_params=pltpu.CompilerParams(needs_layout_passes=False),
)


def _b_kernel(comp_smem, x_ref, bases_ref, root_ref, bias_ref,
              t_ref, rootx_ref, xs_ref):
    r = pl.program_id(0)

    @pl.when(r == 0)
    def _():
        xs_ref[...] = x_ref[...]
        rootx_ref[...] = (
            jnp.dot(xs_ref[...], root_ref[...],
                    preferred_element_type=jnp.float32)
            + bias_ref[...])

    w = comp_smem[r, 0] * bases_ref[0]
    for b in range(1, B):
        w = w + comp_smem[r, b] * bases_ref[b]
    t_ref[0] = jnp.dot(xs_ref[...], w, preferred_element_type=jnp.float32)


def _b2_kernel(comp_smem, acc_ref, rootxp_ref, bases_ref, root_ref, bias_ref,
               t_ref, rootx_ref, xs_ref):
    r = pl.program_id(0)

    @pl.when(r == 0)
    def _():
        xs_ref[...] = jnp.maximum(
            acc_ref[0, :N] + acc_ref[1, :N] + rootxp_ref[...], 0.0)
        rootx_ref[...] = (
            jnp.dot(xs_ref[...], root_ref[...],
                    preferred_element_type=jnp.float32)
            + bias_ref[...])

    w = comp_smem[r, 0] * bases_ref[0]
    for b in range(1, B):
        w = w + comp_smem[r, b] * bases_ref[b]
    t_ref[0] = jnp.dot(xs_ref[...], w, preferred_element_type=jnp.float32)


_B_OUT = [
    pl.BlockSpec((1, N, D), lambda r: (r, 0, 0)),
    pl.BlockSpec((N, D), lambda r: (0, 0)),
]
_B_OUT_SHAPE = [
    jax.ShapeDtypeStruct((R, N, D), jnp.float32),
    jax.ShapeDtypeStruct((N, D), jnp.float32),
]


def _run_b(x, bases, comp, root, bias2):
    return pl.pallas_call(
        _b_kernel,
        grid=(R,),
        in_specs=[
            pl.BlockSpec(memory_space=pltpu.SMEM),
            pl.BlockSpec((N, D), lambda r: (0, 0)),
            pl.BlockSpec((B, D, D), lambda r: (0, 0, 0)),
            pl.BlockSpec((D, D), lambda r: (0, 0)),
            pl.BlockSpec((1, D), lambda r: (0, 0)),
        ],
        out_specs=_B_OUT,
        out_shape=_B_OUT_SHAPE,
        scratch_shapes=[pltpu.VMEM((N, D), jnp.float32)],
    )(comp, x, bases, root, bias2)


def _run_b2(acc, rootxp, bases, comp, root, bias2):
    return pl.pallas_call(
        _b2_kernel,
        grid=(R,),
        in_specs=[
            pl.BlockSpec(memory_space=pltpu.SMEM),
            pl.BlockSpec((NC, NP, D), lambda r: (0, 0, 0)),
            pl.BlockSpec((N, D), lambda r: (0, 0)),
            pl.BlockSpec((B, D, D), lambda r: (0, 0, 0)),
            pl.BlockSpec((D, D), lambda r: (0, 0)),
            pl.BlockSpec((1, D), lambda r: (0, 0)),
        ],
        out_specs=_B_OUT,
        out_shape=_B_OUT_SHAPE,
        scratch_shapes=[pltpu.VMEM((N, D), jnp.float32)],
    )(comp, acc, rootxp, bases, root, bias2)


def _f_kernel(acc_ref, rootx_ref, out_ref):
    out_ref[...] = jnp.maximum(
        acc_ref[0, :N] + acc_ref[1, :N] + rootx_ref[...], 0.0)


def _run_f(acc, rootx):
    return pl.pallas_call(
        _f_kernel,
        out_shape=jax.ShapeDtypeStruct((N, D), jnp.float32),
    )(acc, rootx)


def kernel(edge_index, edge_type, entity_embedding,
           bases0, comp0, root0, bias0,
           bases1, comp1, root1, bias1):
    src = edge_index[0]
    dst = edge_index[1]
    dst3 = dst.reshape(NW, SUP, 2 * CPS, KH)
    znd = jnp.zeros((NP, D), jnp.float32)

    seg, rowid, w = _a1(src, dst, edge_type)
    del seg

    t0, rootx0 = _run_b(entity_embedding, bases0, comp0, root0,
                        bias0.reshape(1, D))
    acc0 = _c(t0.reshape(R * N, D), rowid, dst3, w, znd)
    t1, rootx1 = _run_b2(acc0, rootx0, bases1, comp1, root1,
                         bias1.reshape(1, D))
    acc1 = _c(t1.reshape(R * N, D), rowid, dst3, w, znd)
    return _run_f(acc1, rootx1)


# speedup vs baseline: 1.0066x; 1.0066x over previous
import jax
import jax.numpy as jnp
from jax import lax
from jax.experimental import pallas as pl
from jax.experimental.pallas import tpu as pltpu
from jax.experimental.pallas import tpu_sc as plsc

N = 10000
R = 32
D = 128
B = 25
E = 320000
NR = N * R

NC = 2
NS = 16
NW = NC * NS
EPW = E // NW

KA = 80
NCHA = EPW // KA
NR_PT = NR // NS

KC = 80
KH = KC // 2
NP = 10240
N_PT = NP // NS
SUP = 5
EPS = EPW // SUP
CPS = EPS // KC

_MESH2 = plsc.VectorSubcoreMesh(core_axis_name="c", subcore_axis_name="s")


def _a1_body(src_hbm, dst_hbm, rel_hbm,
             seg_hbm, rowid_hbm, w_hbm,
             src_v, dst_v, rel_v, segf_v, rowf_v, segc_v, ones_v, zb_v,
             cnt_v, cnt2_v, wf_v, cnt_sh, csem, csem2):
    c = lax.axis_index("c")
    s = lax.axis_index("s")

    def zfill(i, _):
        zb_v[pl.ds(i * 16, 16)] = jnp.zeros((16,), jnp.float32)
        return 0

    lax.fori_loop(0, NR_PT // 16, zfill, 0)
    pltpu.sync_copy(zb_v, cnt_sh.at[pl.ds(s * NR_PT, NR_PT)])
    for j in range(KA // 16):
        ones_v[pl.ds(j * 16, 16)] = jnp.ones((16,), jnp.float32)
    plsc.subcore_barrier()

    for h in range(2):
        rng = s + NS * h
        base = rng * EPW
        pltpu.sync_copy(src_hbm.at[pl.ds(base, EPW)], src_v)
        pltpu.sync_copy(dst_hbm.at[pl.ds(base, EPW)], dst_v)
        pltpu.sync_copy(rel_hbm.at[pl.ds(base, EPW)], rel_v)

        def chunk(ci, _):
            for j in range(KA // 16):
                off = ci * KA + j * 16
                s16 = src_v[pl.ds(off, 16)]
                d16 = dst_v[pl.ds(off, 16)]
                r16 = rel_v[pl.ds(off, 16)]
                seg16 = d16 * R + r16
                row16 = r16 * N + s16
                segc_v[pl.ds(j * 16, 16)] = seg16
                segf_v[pl.ds(off, 16)] = seg16
                rowf_v[pl.ds(off, 16)] = row16
            pltpu.sync_copy(ones_v, cnt_sh.at[segc_v], add=True)
            return 0

        lax.fori_loop(0, NCHA, chunk, 0)

        @pl.when(c == h)
        def _():
            pltpu.sync_copy(segf_v, seg_hbm.at[pl.ds(base, EPW)])
            pltpu.sync_copy(rowf_v, rowid_hbm.at[pl.ds(base, EPW)])

    plsc.subcore_barrier()

    own = (c * NS + s) * EPW
    pltpu.sync_copy(seg_hbm.at[pl.ds(own, EPW)], segf_v)

    cbufs = (cnt_v, cnt2_v)
    csems = (csem, csem2)

    def _wissue(ci, b):
        pltpu.async_copy(cnt_sh.at[segf_v.at[pl.ds(ci * KA, KA)]],
                         cbufs[b], csems[b])

    _wissue(0, 0)
    _wissue(1, 1)

    def wpair(i2, _):
        for b in range(2):
            ci = i2 * 2 + b
            pltpu.make_async_copy(cnt_sh.at[segf_v.at[pl.ds(0, KA)]],
                                  cbufs[b], csems[b]).wait()
            nci = ci + 2

            @pl.when(nci < NCHA)
            def _():
                _wissue(nci, b)

            for j in range(KA // 16):
                cv = cbufs[b][pl.ds(j * 16, 16)]
                wf_v[pl.ds(ci * KA + j * 16, 16)] = 1.0 / cv
        return 0

    lax.fori_loop(0, NCHA // 2, wpair, 0)
    pltpu.make_async_copy(cnt_sh.at[segf_v.at[pl.ds(0, KA)]],
                          cbufs[0], csems[0]).wait()
    for j in range(KA // 16):
        cv = cnt_v[pl.ds(j * 16, 16)]
        wf_v[pl.ds((NCHA - 1) * KA + j * 16, 16)] = 1.0 / cv
    pltpu.sync_copy(wf_v, w_hbm.at[pl.ds(own, EPW)])


_a1 = pl.kernel(
    _a1_body,
    out_type=(
        jax.ShapeDtypeStruct((E,), jnp.int32),
        jax.ShapeDtypeStruct((E,), jnp.int32),
        jax.ShapeDtypeStruct((E,), jnp.float32),
    ),
    mesh=_MESH2,
    scratch_types=(
        pltpu.VMEM((EPW,), jnp.int32),
        pltpu.VMEM((EPW,), jnp.int32),
        pltpu.VMEM((EPW,), jnp.int32),
        pltpu.VMEM((EPW,), jnp.int32),
        pltpu.VMEM((EPW,), jnp.int32),
        pltpu.VMEM((KA,), jnp.int32),
        pltpu.VMEM((KA,), jnp.float32),
        pltpu.VMEM((NR_PT,), jnp.float32),
        pltpu.VMEM((KA,), jnp.float32),
        pltpu.VMEM((KA,), jnp.float32),
        pltpu.VMEM((EPW,), jnp.float32),
        pltpu.VMEM_SHARED((NR,), jnp.float32),
        pltpu.SemaphoreType.DMA,
        pltpu.SemaphoreType.DMA,
    ),
)


def _c_body(t_hbm, rowid_hbm, dst3_hbm, w_hbm, znd_hbm,
            acc_hbm,
            rowid_v, w_v, dst2_v, gb0, gb1, sb0, sb1, acc_sh,
            gsem0, gsem1, ssem0, ssem1):
    c = lax.axis_index("c")
    s = lax.axis_index("s")
    wid = c * NS + s
    base = wid * EPW

    pltpu.sync_copy(znd_hbm.at[pl.ds(s * N_PT, N_PT)],
                    acc_sh.at[pl.ds(s * N_PT, N_PT)])
    plsc.subcore_barrier()

    gbufs = (gb0, gb1)
    gsems = (gsem0, gsem1)
    sbufs = (sb0, sb1)
    ssems = (ssem0, ssem1)

    def _issue(ci, b):
        idx = rowid_v.at[pl.ds(ci * KC, KC)]
        pltpu.async_copy(t_hbm.at[idx], gbufs[b], gsems[b])

    def _wait_gather(b):
        pltpu.make_async_copy(t_hbm.at[rowid_v.at[pl.ds(0, KC)]],
                              gbufs[b], gsems[b]).wait()

    def _wait_scatter(h):
        pltpu.make_async_copy(sbufs[h], acc_sh.at[dst2_v.at[0]],
                              ssems[h]).wait()

    def _chunk(ci, b, first):
        _wait_gather(b)
        gbuf = gbufs[b]
        cbase = ci * KC
        w16 = tuple(w_v[pl.ds(cbase + 16 * j, 16)] for j in range(KC // 16))
        for h in range(2):
            if not first:
                _wait_scatter(h)
            sbuf = sbufs[h]
            for el in range(KH):
                e = h * KH + el
                j, lane = e // 16, e % 16
                wb = w16[j].at[jnp.full((16,), lane, jnp.int32)].get(
                    mode="promise_in_bounds")
                for k in range(D // 16):
                    sbuf[el, pl.ds(k * 16, 16)] = (
                        gbuf[e, pl.ds(k * 16, 16)] * wb)
            pltpu.async_copy(sbufs[h], acc_sh.at[dst2_v.at[2 * ci + h]],
                             ssems[h], add=True)

        nci = ci + 2

        @pl.when(nci < CPS)
        def _():
            _issue(nci, b)

    def sup(g, _):
        @pl.when(g > 0)
        def _():
            _wait_scatter(0)
            _wait_scatter(1)

        sbase = base + g * EPS
        pltpu.sync_copy(rowid_hbm.at[pl.ds(sbase, EPS)], rowid_v)
        pltpu.sync_copy(w_hbm.at[pl.ds(sbase, EPS)], w_v)
        pltpu.sync_copy(dst3_hbm.at[wid, g], dst2_v)
        _issue(0, 0)
        _issue(1, 1)

        _chunk(0, 0, True)
        _chunk(1, 1, False)

        def pair(i2, _):
            ci0 = i2 * 2 + 2
            _chunk(ci0, 0, False)
            _chunk(ci0 + 1, 1, False)
            return 0

        lax.fori_loop(0, (CPS - 2) // 2, pair, 0)
        _chunk(CPS - 1, 0, False)
        return 0

    lax.fori_loop(0, SUP, sup, 0)
    _wait_scatter(0)
    _wait_scatter(1)
    plsc.subcore_barrier()
    pltpu.sync_copy(acc_sh.at[pl.ds(s * N_PT, N_PT)],
                    acc_hbm.at[c, pl.ds(s * N_PT, N_PT)])


_c = pl.kernel(
    _c_body,
    out_type=jax.ShapeDtypeStruct((NC, NP, D), jnp.float32),
    mesh=_MESH2,
    scratch_types=(
        pltpu.VMEM((EPS,), jnp.int32),
        pltpu.VMEM((EPS,), jnp.float32),
        pltpu.VMEM((2 * CPS, KH), jnp.int32),
        pltpu.VMEM((KC, D), jnp.float32),
        pltpu.VMEM((KC, D), jnp.float32),
        pltpu.VMEM((KH, D), jnp.float32),
        pltpu.VMEM((KH, D), jnp.float32),
        pltpu.VMEM_SHARED((NP, D), jnp.float32),
        pltpu.SemaphoreType.DMA,
        pltpu.SemaphoreType.DMA,
        pltpu.SemaphoreType.DMA,
        pltpu.SemaphoreType.DMA,
    ),
    compiler_params=pltpu.CompilerParams(needs_layout_passes=False),
)


def _b_kernel(comp_smem, x_ref, bases_ref, root_ref, bias_ref,
              t_ref, rootx_ref, xs_ref):
    r = pl.program_id(0)

    @pl.when(r == 0)
    def _():
        xs_ref[...] = x_ref[...]
        rootx_ref[...] = (
            jnp.dot(xs_ref[...], root_ref[...],
                    preferred_element_type=jnp.float32)
            + bias_ref[...])

    w = comp_smem[r, 0] * bases_ref[0]
    for b in range(1, B):
        w = w + comp_smem[r, b] * bases_ref[b]
    t_ref[0] = jnp.dot(xs_ref[...], w, preferred_element_type=jnp.float32)


def _b2_kernel(comp_smem, acc_ref, rootxp_ref, bases_ref, root_ref, bias_ref,
               t_ref, rootx_ref, xs_ref):
    r = pl.program_id(0)

    @pl.when(r == 0)
    def _():
        xs_ref[...] = jnp.maximum(
            acc_ref[0, :N] + acc_ref[1, :N] + rootxp_ref[...], 0.0)
        rootx_ref[...] = (
            jnp.dot(xs_ref[...], root_ref[...],
                    preferred_element_type=jnp.float32)
            + bias_ref[...])

    w = comp_smem[r, 0] * bases_ref[0]
    for b in range(1, B):
        w = w + comp_smem[r, b] * bases_ref[b]
    t_ref[0] = jnp.dot(xs_ref[...], w, preferred_element_type=jnp.float32)


_B_OUT = [
    pl.BlockSpec((1, N, D), lambda r: (r, 0, 0)),
    pl.BlockSpec((N, D), lambda r: (0, 0)),
]
_B_OUT_SHAPE = [
    jax.ShapeDtypeStruct((R, N, D), jnp.float32),
    jax.ShapeDtypeStruct((N, D), jnp.float32),
]


def _run_b(x, bases, comp, root, bias2):
    return pl.pallas_call(
        _b_kernel,
        grid=(R,),
        in_specs=[
            pl.BlockSpec(memory_space=pltpu.SMEM),
            pl.BlockSpec((N, D), lambda r: (0, 0)),
            pl.BlockSpec((B, D, D), lambda r: (0, 0, 0)),
            pl.BlockSpec((D, D), lambda r: (0, 0)),
            pl.BlockSpec((1, D), lambda r: (0, 0)),
        ],
        out_specs=_B_OUT,
        out_shape=_B_OUT_SHAPE,
        scratch_shapes=[pltpu.VMEM((N, D), jnp.float32)],
    )(comp, x, bases, root, bias2)


def _run_b2(acc, rootxp, bases, comp, root, bias2):
    return pl.pallas_call(
        _b2_kernel,
        grid=(R,),
        in_specs=[
            pl.BlockSpec(memory_space=pltpu.SMEM),
            pl.BlockSpec((NC, NP, D), lambda r: (0, 0, 0)),
            pl.BlockSpec((N, D), lambda r: (0, 0)),
            pl.BlockSpec((B, D, D), lambda r: (0, 0, 0)),
            pl.BlockSpec((D, D), lambda r: (0, 0)),
            pl.BlockSpec((1, D), lambda r: (0, 0)),
        ],
        out_specs=_B_OUT,
        out_shape=_B_OUT_SHAPE,
        scratch_shapes=[pltpu.VMEM((N, D), jnp.float32)],
    )(comp, acc, rootxp, bases, root, bias2)


def _f_kernel(acc_ref, rootx_ref, out_ref):
    out_ref[...] = jnp.maximum(
        acc_ref[0, :N] + acc_ref[1, :N] + rootx_ref[...], 0.0)


def _run_f(acc, rootx):
    return pl.pallas_call(
        _f_kernel,
        out_shape=jax.ShapeDtypeStruct((N, D), jnp.float32),
    )(acc, rootx)


def kernel(edge_index, edge_type, entity_embedding,
           bases0, comp0, root0, bias0,
           bases1, comp1, root1, bias1):
    src = edge_index[0]
    dst = edge_index[1]
    dst3 = dst.reshape(NW, SUP, 2 * CPS, KH)
    znd = jnp.zeros((NP, D), jnp.float32)

    seg, rowid, w = _a1(src, dst, edge_type)
    del seg

    t0, rootx0 = _run_b(entity_embedding, bases0, comp0, root0,
                        bias0.reshape(1, D))
    acc0 = _c(t0.reshape(R * N, D), rowid, dst3, w, znd)
    t1, rootx1 = _run_b2(acc0, rootx0, bases1, comp1, root1,
                         bias1.reshape(1, D))
    acc1 = _c(t1.reshape(R * N, D), rowid, dst3, w, znd)
    return _run_f(acc1, rootx1)
